# baseline (speedup 1.0000x reference)
import functools

import jax
import jax.numpy as jnp
from jax import lax
from jax.experimental import pallas as pl
from jax.experimental.pallas import tpu as pltpu
from jax.experimental.pallas import tpu_sc as plsc

_N = 10000
_E = 320000
_D = 128
_KAPPA = 0.95

_NC = 2
_NS = 16
_NW = _NC * _NS
_CHUNK = 128
_NCHUNK = 80
_EPW = _NCHUNK * _CHUNK
_EPAD = _NW * _EPW
_NP = 10240
_RPS = _NP // _NS
_ZROWS = 128
_PADIDX = _NP - 1

_mesh = plsc.VectorSubcoreMesh(core_axis_name="c", subcore_axis_name="s")


@functools.partial(
    pl.kernel,
    out_type=jax.ShapeDtypeStruct((_NC, _NP, 16), jnp.float32),
    mesh=_mesh,
    scratch_types=[
        pltpu.VMEM((_NCHUNK, _CHUNK), jnp.int32),
        pltpu.VMEM((_CHUNK, 16), jnp.float32),
        pltpu.VMEM((_ZROWS, 16), jnp.float32),
        pltpu.VMEM_SHARED((_NP, 16), jnp.float32),
    ],
)
def _sc_degree(dst_hbm, out_hbm, didx_v, ones_v, z_v, acc_sh):
    c = lax.axis_index("c")
    s = lax.axis_index("s")

    @pl.loop(0, _CHUNK)
    def _(r):
        ones_v.at[pl.ds(r, 1), pl.ds(0, 16)][...] = jnp.ones((1, 16), jnp.float32)

    @pl.loop(0, _ZROWS)
    def _(r):
        z_v.at[pl.ds(r, 1), pl.ds(0, 16)][...] = jnp.zeros((1, 16), jnp.float32)

    @pl.loop(0, _RPS // _ZROWS)
    def _(k):
        pltpu.sync_copy(z_v, acc_sh.at[pl.ds(s * _RPS + k * _ZROWS, _ZROWS)])

    plsc.subcore_barrier()
    w = c * _NS + s
    pltpu.sync_copy(dst_hbm.at[w], didx_v)

    @pl.loop(0, _NCHUNK)
    def _(i):
        pltpu.sync_copy(ones_v, acc_sh.at[didx_v.at[i]], add=True)

    plsc.subcore_barrier()
    pltpu.sync_copy(acc_sh.at[pl.ds(s * _RPS, _RPS)],
                    out_hbm.at[c, pl.ds(s * _RPS, _RPS)])


_HCHUNK = _NCHUNK // 2


@functools.partial(
    pl.kernel,
    out_type=jax.ShapeDtypeStruct((_NC, _NP, _D), jnp.float32),
    mesh=_mesh,
    scratch_types=[
        pltpu.VMEM((_HCHUNK, _CHUNK), jnp.int32),
        pltpu.VMEM((_HCHUNK, _CHUNK), jnp.int32),
        pltpu.VMEM((_CHUNK, _D), jnp.float32),
        pltpu.VMEM((_CHUNK, _D), jnp.float32),
        pltpu.VMEM_SHARED((_NP, _D), jnp.float32),
        pltpu.SemaphoreType.DMA,
        pltpu.SemaphoreType.DMA,
    ],
)
def _sc_messages(g_hbm, src_hbm, dst_hbm, out_hbm,
                 sidx_v, didx_v, rows0_v, rows1_v, acc_sh, sem0, sem1):
    c = lax.axis_index("c")
    s = lax.axis_index("s")

    @pl.loop(0, _CHUNK)
    def _(r):
        @pl.loop(0, _D, step=16)
        def _(cc):
            rows0_v.at[pl.ds(r, 1), pl.ds(cc, 16)][...] = jnp.zeros((1, 16), jnp.float32)

    @pl.loop(0, _RPS // _CHUNK)
    def _(k):
        pltpu.sync_copy(rows0_v, acc_sh.at[pl.ds(s * _RPS + k * _CHUNK, _CHUNK)])

    plsc.subcore_barrier()
    w = c * _NS + s
    bufs = ((rows0_v, sem0), (rows1_v, sem1))

    for h in range(2):
        pltpu.sync_copy(src_hbm.at[w, pl.ds(h * _HCHUNK, _HCHUNK)], sidx_v)
        pltpu.sync_copy(dst_hbm.at[w, pl.ds(h * _HCHUNK, _HCHUNK)], didx_v)

        pltpu.async_copy(g_hbm.at[sidx_v.at[0]], rows0_v, sem0)
        pltpu.async_copy(g_hbm.at[sidx_v.at[1]], rows1_v, sem1)

        @pl.loop(0, _HCHUNK, step=2)
        def _(i):
            for b, (rows_v, sem) in enumerate(bufs):
                ci = i + b
                pltpu.make_async_copy(g_hbm.at[sidx_v.at[ci]], rows_v, sem).wait()
                pltpu.sync_copy(rows_v, acc_sh.at[didx_v.at[ci]], add=True)

                @pl.when(ci + 2 < _HCHUNK)
                def _():
                    pltpu.async_copy(g_hbm.at[sidx_v.at[ci + 2]], rows_v, sem)

    plsc.subcore_barrier()
    pltpu.sync_copy(acc_sh.at[pl.ds(s * _RPS, _RPS)],
                    out_hbm.at[c, pl.ds(s * _RPS, _RPS)])


_BLK = 2000
_GRID = _N // _BLK


def _tc_prep_body(wc_ref, wm_ref, emb_ref, feat_ref, h_ref, dense_ref):
    wc = wc_ref[...]
    rs = jnp.sum(jnp.abs(wc), axis=1, keepdims=True)
    scale = jnp.where(rs > _KAPPA, _KAPPA / rs, 1.0)
    wproj = wc * scale
    h_ref[...] = jnp.dot(emb_ref[...], wproj.T,
                         preferred_element_type=jnp.float32,
                         precision=lax.Precision.HIGHEST)
    dense_ref[...] = jnp.dot(feat_ref[...], wm_ref[...].T,
                             preferred_element_type=jnp.float32,
                             precision=lax.Precision.HIGHEST)


def _tc_prep(W_conv, W_mlp, emb, feat):
    return pl.pallas_call(
        _tc_prep_body,
        grid=(_GRID,),
        in_specs=[
            pl.BlockSpec((_D, _D), lambda i: (0, 0)),
            pl.BlockSpec((_D, _D), lambda i: (0, 0)),
            pl.BlockSpec((_BLK, _D), lambda i: (i, 0)),
            pl.BlockSpec((_BLK, _D), lambda i: (i, 0)),
        ],
        out_specs=[
            pl.BlockSpec((_BLK, _D), lambda i: (i, 0)),
            pl.BlockSpec((_BLK, _D), lambda i: (i, 0)),
        ],
        out_shape=[
            jax.ShapeDtypeStruct((_N, _D), jnp.float32),
            jax.ShapeDtypeStruct((_N, _D), jnp.float32),
        ],
    )(W_conv, W_mlp, emb, feat)


_BLKP = 2048
_GRIDP = _NP // _BLKP


def _tc_scale_body(h_ref, degp_ref, g_ref):
    rid = pl.program_id(0) * _BLKP + lax.broadcasted_iota(jnp.int32, (_BLKP, 1), 0)
    deg = degp_ref[0, :, 0:1] + degp_ref[1, :, 0:1] + 1.0
    g_ref[...] = jnp.where(rid < _N, h_ref[...] * lax.rsqrt(deg), 0.0)


def _tc_scale(h, degp):
    return pl.pallas_call(
        _tc_scale_body,
        grid=(_GRIDP,),
        in_specs=[
            pl.BlockSpec((_BLKP, _D), lambda i: (i, 0)),
            pl.BlockSpec((_NC, _BLKP, 16), lambda i: (0, i, 0)),
        ],
        out_specs=pl.BlockSpec((_BLKP, _D), lambda i: (i, 0)),
        out_shape=jax.ShapeDtypeStruct((_NP, _D), jnp.float32),
    )(h, degp)


def _tc_final_body(p_ref, g_ref, dense_ref, degp_ref, o_ref):
    deg = degp_ref[0, :, 0:1] + degp_ref[1, :, 0:1] + 1.0
    dinv = lax.rsqrt(deg)
    acc = p_ref[0] + p_ref[1] + g_ref[...]
    o_ref[...] = jnp.maximum(acc * dinv + dense_ref[...], 0.0)


def _tc_final(parts, g, dense, degp):
    return pl.pallas_call(
        _tc_final_body,
        grid=(_GRID,),
        in_specs=[
            pl.BlockSpec((_NC, _BLK, _D), lambda i: (0, i, 0)),
            pl.BlockSpec((_BLK, _D), lambda i: (i, 0)),
            pl.BlockSpec((_BLK, _D), lambda i: (i, 0)),
            pl.BlockSpec((_NC, _BLK, 16), lambda i: (0, i, 0)),
        ],
        out_specs=pl.BlockSpec((_BLK, _D), lambda i: (i, 0)),
        out_shape=jax.ShapeDtypeStruct((_N, _D), jnp.float32),
    )(parts, g, dense, degp)


def kernel(features, sparse_adj, W_conv, W_mlp, embeddings):
    pad = jnp.full((2, _EPAD - _E), _PADIDX, dtype=sparse_adj.dtype)
    adj = jnp.concatenate([sparse_adj, pad], axis=1)
    src = adj[0].reshape(_NW, _NCHUNK, _CHUNK)
    dst = adj[1].reshape(_NW, _NCHUNK, _CHUNK)
    degp = _sc_degree(dst)
    h, dense = _tc_prep(W_conv, W_mlp, embeddings, features)
    g = _tc_scale(h, degp)
    parts = _sc_messages(g, src, dst)
    return _tc_final(parts, g, dense, degp)


# speedup vs baseline: 1.7499x; 1.7499x over previous
import functools

import jax
import jax.numpy as jnp
from jax import lax
from jax.experimental import pallas as pl
from jax.experimental.pallas import tpu as pltpu
from jax.experimental.pallas import tpu_sc as plsc

_N = 10000
_E = 320000
_D = 128
_KAPPA = 0.95

_NC = 2
_NS = 16
_NW = _NC * _NS
_CHUNK = 128
_NCHUNK = 80
_EPW = _NCHUNK * _CHUNK
_EPAD = _NW * _EPW
_NP = 10240
_RPS = _NP // _NS
_ZROWS = 128

_mesh = plsc.VectorSubcoreMesh(core_axis_name="c", subcore_axis_name="s")


@functools.partial(
    pl.kernel,
    out_type=jax.ShapeDtypeStruct((_NC, _NP, _D), jnp.float32),
    mesh=_mesh,
    scratch_types=[
        pltpu.VMEM((_NCHUNK, _CHUNK), jnp.int32),
        pltpu.VMEM((_CHUNK, _D), jnp.float32),
        pltpu.VMEM_SHARED((_NP, _D), jnp.float32),
    ],
)
def _sc_degree(dst_hbm, out_hbm, didx_v, ones_v, acc_sh):
    c = lax.axis_index("c")
    s = lax.axis_index("s")

    @pl.loop(0, _CHUNK)
    def _(r):
        @pl.loop(0, _D, step=16)
        def _(cc):
            ones_v.at[pl.ds(r, 1), pl.ds(cc, 16)][...] = jnp.zeros((1, 16), jnp.float32)

    @pl.loop(0, _RPS // _CHUNK)
    def _(k):
        pltpu.sync_copy(ones_v, acc_sh.at[pl.ds(s * _RPS + k * _CHUNK, _CHUNK)])

    @pl.loop(0, _CHUNK)
    def _(r):
        @pl.loop(0, _D, step=16)
        def _(cc):
            ones_v.at[pl.ds(r, 1), pl.ds(cc, 16)][...] = jnp.ones((1, 16), jnp.float32)

    w = c * _NS + s
    pltpu.sync_copy(dst_hbm.at[w], didx_v)
    plsc.subcore_barrier()

    @pl.loop(0, _NCHUNK)
    def _(i):
        pltpu.sync_copy(ones_v, acc_sh.at[didx_v.at[i]], add=True)

    plsc.subcore_barrier()
    @pl.loop(0, _RPS // _CHUNK)
    def _(k):
        off = s * _RPS + k * _CHUNK
        pltpu.sync_copy(acc_sh.at[pl.ds(off, _CHUNK)], ones_v)
        pltpu.sync_copy(ones_v, out_hbm.at[c, pl.ds(off, _CHUNK)])


_HCHUNK = _NCHUNK // 2


@functools.partial(
    pl.kernel,
    out_type=jax.ShapeDtypeStruct((_NC, _NP, _D), jnp.float32),
    mesh=_mesh,
    scratch_types=[
        pltpu.VMEM((_HCHUNK, _CHUNK), jnp.int32),
        pltpu.VMEM((_HCHUNK, _CHUNK), jnp.int32),
        pltpu.VMEM((_CHUNK, _D), jnp.float32),
        pltpu.VMEM((_CHUNK, _D), jnp.float32),
        pltpu.VMEM_SHARED((_NP, _D), jnp.float32),
        pltpu.SemaphoreType.DMA,
        pltpu.SemaphoreType.DMA,
    ],
)
def _sc_messages(g_hbm, src_hbm, dst_hbm, out_hbm,
                 sidx_v, didx_v, rows0_v, rows1_v, acc_sh, sem0, sem1):
    c = lax.axis_index("c")
    s = lax.axis_index("s")

    @pl.loop(0, _CHUNK)
    def _(r):
        @pl.loop(0, _D, step=16)
        def _(cc):
            rows0_v.at[pl.ds(r, 1), pl.ds(cc, 16)][...] = jnp.zeros((1, 16), jnp.float32)

    @pl.loop(0, _RPS // _CHUNK)
    def _(k):
        pltpu.sync_copy(rows0_v, acc_sh.at[pl.ds(s * _RPS + k * _CHUNK, _CHUNK)])

    w = c * _NS + s
    pltpu.sync_copy(src_hbm.at[w, pl.ds(0, _HCHUNK)], sidx_v)
    pltpu.sync_copy(dst_hbm.at[w, pl.ds(0, _HCHUNK)], didx_v)
    plsc.subcore_barrier()

    for h in range(2):
        if h:
            pltpu.sync_copy(src_hbm.at[w, pl.ds(h * _HCHUNK, _HCHUNK)], sidx_v)
            pltpu.sync_copy(dst_hbm.at[w, pl.ds(h * _HCHUNK, _HCHUNK)], didx_v)

        @pl.loop(0, _HCHUNK)
        def _(i):
            pltpu.async_copy(g_hbm.at[sidx_v.at[i]], rows0_v, sem0).wait()
            pltpu.sync_copy(rows0_v, acc_sh.at[didx_v.at[i]], add=True)

    plsc.subcore_barrier()
    @pl.loop(0, _RPS // _CHUNK)
    def _(k):
        off = s * _RPS + k * _CHUNK
        pltpu.sync_copy(acc_sh.at[pl.ds(off, _CHUNK)], rows1_v)
        pltpu.sync_copy(rows1_v, out_hbm.at[c, pl.ds(off, _CHUNK)])


_BLK = 2000
_GRID = _N // _BLK


def _tc_prep_body(wc_ref, wm_ref, emb_ref, feat_ref, h_ref, dense_ref):
    wc = wc_ref[...]
    rs = jnp.sum(jnp.abs(wc), axis=1, keepdims=True)
    scale = jnp.where(rs > _KAPPA, _KAPPA / rs, 1.0)
    wproj = wc * scale
    h_ref[...] = jnp.dot(emb_ref[...], wproj.T,
                         preferred_element_type=jnp.float32,
                         precision=lax.Precision.HIGHEST)
    dense_ref[...] = jnp.dot(feat_ref[...], wm_ref[...].T,
                             preferred_element_type=jnp.float32,
                             precision=lax.Precision.HIGHEST)


def _tc_prep(W_conv, W_mlp, emb, feat):
    return pl.pallas_call(
        _tc_prep_body,
        grid=(_GRID,),
        in_specs=[
            pl.BlockSpec((_D, _D), lambda i: (0, 0)),
            pl.BlockSpec((_D, _D), lambda i: (0, 0)),
            pl.BlockSpec((_BLK, _D), lambda i: (i, 0)),
            pl.BlockSpec((_BLK, _D), lambda i: (i, 0)),
        ],
        out_specs=[
            pl.BlockSpec((_BLK, _D), lambda i: (i, 0)),
            pl.BlockSpec((_BLK, _D), lambda i: (i, 0)),
        ],
        out_shape=[
            jax.ShapeDtypeStruct((_N, _D), jnp.float32),
            jax.ShapeDtypeStruct((_N, _D), jnp.float32),
        ],
    )(W_conv, W_mlp, emb, feat)


_BLKP = 2048
_GRIDP = _NP // _BLKP


def _tc_scale_body(h_ref, degp_ref, g_ref):
    rid = pl.program_id(0) * _BLKP + lax.broadcasted_iota(jnp.int32, (_BLKP, 1), 0)
    deg = degp_ref[0, :, 0:1] + degp_ref[1, :, 0:1] + 1.0
    g_ref[...] = jnp.where(rid < _N, h_ref[...] * lax.rsqrt(deg), 0.0)


def _tc_scale(h, degp):
    return pl.pallas_call(
        _tc_scale_body,
        grid=(_GRIDP,),
        in_specs=[
            pl.BlockSpec((_BLKP, _D), lambda i: (i, 0)),
            pl.BlockSpec((_NC, _BLKP, _D), lambda i: (0, i, 0)),
        ],
        out_specs=pl.BlockSpec((_BLKP, _D), lambda i: (i, 0)),
        out_shape=jax.ShapeDtypeStruct((_NP, _D), jnp.float32),
    )(h, degp)


def _tc_final_body(p_ref, g_ref, dense_ref, degp_ref, o_ref):
    deg = degp_ref[0, :, 0:1] + degp_ref[1, :, 0:1] + 1.0
    dinv = lax.rsqrt(deg)
    acc = p_ref[0] + p_ref[1] + g_ref[...]
    o_ref[...] = jnp.maximum(acc * dinv + dense_ref[...], 0.0)


def _tc_final(parts, g, dense, degp):
    return pl.pallas_call(
        _tc_final_body,
        grid=(_GRID,),
        in_specs=[
            pl.BlockSpec((_NC, _BLK, _D), lambda i: (0, i, 0)),
            pl.BlockSpec((_BLK, _D), lambda i: (i, 0)),
            pl.BlockSpec((_BLK, _D), lambda i: (i, 0)),
            pl.BlockSpec((_NC, _BLK, _D), lambda i: (0, i, 0)),
        ],
        out_specs=pl.BlockSpec((_BLK, _D), lambda i: (i, 0)),
        out_shape=jax.ShapeDtypeStruct((_N, _D), jnp.float32),
    )(parts, g, dense, degp)


def kernel(features, sparse_adj, W_conv, W_mlp, embeddings):
    eprw = _E // _NW
    nprw = _EPW - eprw
    padrow = (jnp.arange(nprw, dtype=sparse_adj.dtype) % (_NP - _N)) + _N
    pad = jnp.broadcast_to(padrow, (_NW, nprw))
    src = jnp.concatenate([sparse_adj[0].reshape(_NW, eprw), pad], axis=1)
    dst = jnp.concatenate([sparse_adj[1].reshape(_NW, eprw), pad], axis=1)
    src = src.reshape(_NW, _NCHUNK, _CHUNK)
    dst = dst.reshape(_NW, _NCHUNK, _CHUNK)
    degp = _sc_degree(dst)
    h, dense = _tc_prep(W_conv, W_mlp, embeddings, features)
    g = _tc_scale(h, degp)
    parts = _sc_messages(g, src, dst)
    return _tc_final(parts, g, dense, degp)


# speedup vs baseline: 2.2653x; 1.2945x over previous
import functools

import jax
import jax.numpy as jnp
from jax import lax
from jax.experimental import pallas as pl
from jax.experimental.pallas import tpu as pltpu
from jax.experimental.pallas import tpu_sc as plsc

_N = 10000
_E = 320000
_D = 128
_KAPPA = 0.95

_NC = 2
_NS = 16
_NW = _NC * _NS
_CHUNK = 128
_NCHUNK = 80
_EPW = _NCHUNK * _CHUNK
_EPAD = _NW * _EPW
_NP = 10240
_RPS = _NP // _NS
_ZROWS = 128

_mesh = plsc.VectorSubcoreMesh(core_axis_name="c", subcore_axis_name="s")


@functools.partial(
    pl.kernel,
    out_type=jax.ShapeDtypeStruct((_NC, _NP, _D), jnp.float32),
    mesh=_mesh,
    scratch_types=[
        pltpu.VMEM((_NCHUNK, _CHUNK), jnp.int32),
        pltpu.VMEM((_CHUNK, _D), jnp.float32),
        pltpu.VMEM_SHARED((_NP, _D), jnp.float32),
    ],
)
def _sc_degree(dst_hbm, out_hbm, didx_v, ones_v, acc_sh):
    c = lax.axis_index("c")
    s = lax.axis_index("s")

    @pl.loop(0, _CHUNK)
    def _(r):
        @pl.loop(0, _D, step=16)
        def _(cc):
            ones_v.at[pl.ds(r, 1), pl.ds(cc, 16)][...] = jnp.zeros((1, 16), jnp.float32)

    @pl.loop(0, _RPS // _CHUNK)
    def _(k):
        pltpu.sync_copy(ones_v, acc_sh.at[pl.ds(s * _RPS + k * _CHUNK, _CHUNK)])

    @pl.loop(0, _CHUNK)
    def _(r):
        @pl.loop(0, _D, step=16)
        def _(cc):
            ones_v.at[pl.ds(r, 1), pl.ds(cc, 16)][...] = jnp.ones((1, 16), jnp.float32)

    w = c * _NS + s
    pltpu.sync_copy(dst_hbm.at[w], didx_v)
    plsc.subcore_barrier()

    @pl.loop(0, _NCHUNK)
    def _(i):
        pltpu.sync_copy(ones_v, acc_sh.at[didx_v.at[i]], add=True)

    plsc.subcore_barrier()
    @pl.loop(0, _RPS // _CHUNK)
    def _(k):
        off = s * _RPS + k * _CHUNK
        pltpu.sync_copy(acc_sh.at[pl.ds(off, _CHUNK)], ones_v)
        pltpu.sync_copy(ones_v, out_hbm.at[c, pl.ds(off, _CHUNK)])


_HCHUNK = _NCHUNK // 2


@functools.partial(
    pl.kernel,
    out_type=jax.ShapeDtypeStruct((_NC, _NP, _D), jnp.float32),
    mesh=_mesh,
    scratch_types=[
        pltpu.VMEM((_HCHUNK, _CHUNK), jnp.int32),
        pltpu.VMEM((_HCHUNK, _CHUNK), jnp.int32),
        pltpu.VMEM((_CHUNK, _D), jnp.float32),
        pltpu.VMEM((_CHUNK, _D), jnp.float32),
        pltpu.VMEM_SHARED((_NP, _D), jnp.float32),
        pltpu.SemaphoreType.DMA,
        pltpu.SemaphoreType.DMA,
    ],
)
def _sc_messages(g_hbm, src_hbm, dst_hbm, out_hbm,
                 sidx_v, didx_v, rows0_v, rows1_v, acc_sh, sem0, sem1):
    c = lax.axis_index("c")
    s = lax.axis_index("s")

    @pl.loop(0, _CHUNK)
    def _(r):
        @pl.loop(0, _D, step=16)
        def _(cc):
            rows0_v.at[pl.ds(r, 1), pl.ds(cc, 16)][...] = jnp.zeros((1, 16), jnp.float32)

    @pl.loop(0, _RPS // _CHUNK)
    def _(k):
        pltpu.sync_copy(rows0_v, acc_sh.at[pl.ds(s * _RPS + k * _CHUNK, _CHUNK)])

    w = c * _NS + s
    pltpu.sync_copy(src_hbm.at[w, pl.ds(0, _HCHUNK)], sidx_v)
    pltpu.sync_copy(dst_hbm.at[w, pl.ds(0, _HCHUNK)], didx_v)
    plsc.subcore_barrier()

    bufs = ((rows0_v, sem0), (rows1_v, sem1))
    for h in range(2):
        if h:
            pltpu.sync_copy(src_hbm.at[w, pl.ds(h * _HCHUNK, _HCHUNK)], sidx_v)
            pltpu.sync_copy(dst_hbm.at[w, pl.ds(h * _HCHUNK, _HCHUNK)], didx_v)

        pltpu.async_copy(g_hbm.at[sidx_v.at[0]], rows0_v, sem0)
        pltpu.async_copy(g_hbm.at[sidx_v.at[1]], rows1_v, sem1)

        @pl.loop(0, _HCHUNK, step=2)
        def _(i):
            for b, (rows_v, sem) in enumerate(bufs):
                ci = i + b
                pltpu.make_async_copy(g_hbm.at[sidx_v.at[ci]], rows_v, sem).wait()
                pltpu.sync_copy(rows_v, acc_sh.at[didx_v.at[ci]], add=True)

                @pl.when(ci + 2 < _HCHUNK)
                def _():
                    pltpu.async_copy(g_hbm.at[sidx_v.at[ci + 2]], rows_v, sem)

    plsc.subcore_barrier()
    @pl.loop(0, _RPS // _CHUNK)
    def _(k):
        off = s * _RPS + k * _CHUNK
        pltpu.sync_copy(acc_sh.at[pl.ds(off, _CHUNK)], rows1_v)
        pltpu.sync_copy(rows1_v, out_hbm.at[c, pl.ds(off, _CHUNK)])


_BLK = 2000
_GRID = _N // _BLK


def _tc_prep_body(wc_ref, wm_ref, emb_ref, feat_ref, h_ref, dense_ref):
    wc = wc_ref[...]
    rs = jnp.sum(jnp.abs(wc), axis=1, keepdims=True)
    scale = jnp.where(rs > _KAPPA, _KAPPA / rs, 1.0)
    wproj = wc * scale
    h_ref[...] = jnp.dot(emb_ref[...], wproj.T,
                         preferred_element_type=jnp.float32,
                         precision=lax.Precision.HIGHEST)
    dense_ref[...] = jnp.dot(feat_ref[...], wm_ref[...].T,
                             preferred_element_type=jnp.float32,
                             precision=lax.Precision.HIGHEST)


def _tc_prep(W_conv, W_mlp, emb, feat):
    return pl.pallas_call(
        _tc_prep_body,
        grid=(_GRID,),
        in_specs=[
            pl.BlockSpec((_D, _D), lambda i: (0, 0)),
            pl.BlockSpec((_D, _D), lambda i: (0, 0)),
            pl.BlockSpec((_BLK, _D), lambda i: (i, 0)),
            pl.BlockSpec((_BLK, _D), lambda i: (i, 0)),
        ],
        out_specs=[
            pl.BlockSpec((_BLK, _D), lambda i: (i, 0)),
            pl.BlockSpec((_BLK, _D), lambda i: (i, 0)),
        ],
        out_shape=[
            jax.ShapeDtypeStruct((_N, _D), jnp.float32),
            jax.ShapeDtypeStruct((_N, _D), jnp.float32),
        ],
    )(W_conv, W_mlp, emb, feat)


_BLKP = 2048
_GRIDP = _NP // _BLKP


def _tc_scale_body(h_ref, degp_ref, g_ref):
    rid = pl.program_id(0) * _BLKP + lax.broadcasted_iota(jnp.int32, (_BLKP, 1), 0)
    deg = degp_ref[0, :, 0:1] + degp_ref[1, :, 0:1] + 1.0
    g_ref[...] = jnp.where(rid < _N, h_ref[...] * lax.rsqrt(deg), 0.0)


def _tc_scale(h, degp):
    return pl.pallas_call(
        _tc_scale_body,
        grid=(_GRIDP,),
        in_specs=[
            pl.BlockSpec((_BLKP, _D), lambda i: (i, 0)),
            pl.BlockSpec((_NC, _BLKP, _D), lambda i: (0, i, 0)),
        ],
        out_specs=pl.BlockSpec((_BLKP, _D), lambda i: (i, 0)),
        out_shape=jax.ShapeDtypeStruct((_NP, _D), jnp.float32),
    )(h, degp)


def _tc_final_body(p_ref, g_ref, dense_ref, degp_ref, o_ref):
    deg = degp_ref[0, :, 0:1] + degp_ref[1, :, 0:1] + 1.0
    dinv = lax.rsqrt(deg)
    acc = p_ref[0] + p_ref[1] + g_ref[...]
    o_ref[...] = jnp.maximum(acc * dinv + dense_ref[...], 0.0)


def _tc_final(parts, g, dense, degp):
    return pl.pallas_call(
        _tc_final_body,
        grid=(_GRID,),
        in_specs=[
            pl.BlockSpec((_NC, _BLK, _D), lambda i: (0, i, 0)),
            pl.BlockSpec((_BLK, _D), lambda i: (i, 0)),
            pl.BlockSpec((_BLK, _D), lambda i: (i, 0)),
            pl.BlockSpec((_NC, _BLK, _D), lambda i: (0, i, 0)),
        ],
        out_specs=pl.BlockSpec((_BLK, _D), lambda i: (i, 0)),
        out_shape=jax.ShapeDtypeStruct((_N, _D), jnp.float32),
    )(parts, g, dense, degp)


def kernel(features, sparse_adj, W_conv, W_mlp, embeddings):
    eprw = _E // _NW
    nprw = _EPW - eprw
    padrow = (jnp.arange(nprw, dtype=sparse_adj.dtype) % (_NP - _N)) + _N
    pad = jnp.broadcast_to(padrow, (_NW, nprw))
    src = jnp.concatenate([sparse_adj[0].reshape(_NW, eprw), pad], axis=1)
    dst = jnp.concatenate([sparse_adj[1].reshape(_NW, eprw), pad], axis=1)
    src = src.reshape(_NW, _NCHUNK, _CHUNK)
    dst = dst.reshape(_NW, _NCHUNK, _CHUNK)
    degp = _sc_degree(dst)
    h, dense = _tc_prep(W_conv, W_mlp, embeddings, features)
    g = _tc_scale(h, degp)
    parts = _sc_messages(g, src, dst)
    return _tc_final(parts, g, dense, degp)
